# baseline (speedup 1.0000x reference)
import jax
import jax.numpy as jnp
from jax.experimental import pallas as pl

_THRESHOLD = 0.5
_NB_AVERAGE = 9
_OFFSET = _NB_AVERAGE // 2


def _row_body(x_ref, fb_ref, f_ref, c_ref):
    x = x_ref[...]
    fb = fb_ref[0]
    n_bins = x.shape[-1]
    start_max = n_bins - _NB_AVERAGE

    m = jnp.max(x, axis=-1, keepdims=True)
    iota = jax.lax.broadcasted_iota(jnp.int32, x.shape, 1)
    center = jnp.min(jnp.where(x == m, iota, n_bins), axis=-1, keepdims=True)
    start = jnp.clip(center - _OFFSET, 0, start_max)

    off = (iota - start).astype(jnp.uint32)
    w = jnp.where(off < _NB_AVERAGE, x, 0.0)
    wsum = jnp.sum(w, axis=-1, keepdims=True)
    psum = jnp.sum(w * fb[None, :], axis=-1, keepdims=True)

    f = psum / wsum
    voiced = m > _THRESHOLD
    f_ref[...] = jnp.where(voiced, f, 0.0)
    c_ref[...] = jnp.where(voiced, m, 1.0 - m)


def kernel(inputs, fbins):
    b, t, n_bins = inputs.shape
    rows = b * t
    x = inputs.reshape(rows, n_bins)
    fb = fbins.reshape(1, n_bins)

    blk = 512
    grid = (rows // blk,)
    f, c = pl.pallas_call(
        _row_body,
        grid=grid,
        in_specs=[
            pl.BlockSpec((blk, n_bins), lambda i: (i, 0)),
            pl.BlockSpec((1, n_bins), lambda i: (0, 0)),
        ],
        out_specs=[
            pl.BlockSpec((blk, 1), lambda i: (i, 0)),
            pl.BlockSpec((blk, 1), lambda i: (i, 0)),
        ],
        out_shape=[
            jax.ShapeDtypeStruct((rows, 1), jnp.float32),
            jax.ShapeDtypeStruct((rows, 1), jnp.float32),
        ],
    )(x, fb)
    return jnp.stack([f.reshape(b, t), c.reshape(b, t)], axis=2)


# speedup vs baseline: 1.0315x; 1.0315x over previous
import jax
import jax.numpy as jnp
from jax.experimental import pallas as pl

_THRESHOLD = 0.5
_NB_AVERAGE = 9
_OFFSET = _NB_AVERAGE // 2


def _row_body(x_ref, fb_ref, f_ref, c_ref):
    x = x_ref[...]
    fb = fb_ref[0]
    n_bins = x.shape[-1]
    start_max = n_bins - _NB_AVERAGE

    m = jnp.max(x, axis=-1, keepdims=True)
    iota = jax.lax.broadcasted_iota(jnp.int32, x.shape, 1)
    center = jnp.min(jnp.where(x == m, iota, n_bins), axis=-1, keepdims=True)
    start = jnp.clip(center - _OFFSET, 0, start_max)

    off = (iota - start).astype(jnp.uint32)
    w = jnp.where(off < _NB_AVERAGE, x, 0.0)
    wsum = jnp.sum(w, axis=-1, keepdims=True)
    psum = jnp.sum(w * fb[None, :], axis=-1, keepdims=True)

    f = psum / wsum
    voiced = m > _THRESHOLD
    f_ref[...] = jnp.where(voiced, f, 0.0)[:, 0]
    c_ref[...] = jnp.where(voiced, m, 1.0 - m)[:, 0]


def kernel(inputs, fbins):
    b, t, n_bins = inputs.shape
    rows = b * t
    x = inputs.reshape(rows, n_bins)
    fb = fbins.reshape(1, n_bins)

    blk = 512
    grid = (rows // blk,)
    f, c = pl.pallas_call(
        _row_body,
        grid=grid,
        in_specs=[
            pl.BlockSpec((blk, n_bins), lambda i: (i, 0)),
            pl.BlockSpec((1, n_bins), lambda i: (0, 0)),
        ],
        out_specs=[
            pl.BlockSpec((blk,), lambda i: (i,)),
            pl.BlockSpec((blk,), lambda i: (i,)),
        ],
        out_shape=[
            jax.ShapeDtypeStruct((rows,), jnp.float32),
            jax.ShapeDtypeStruct((rows,), jnp.float32),
        ],
    )(x, fb)
    return jnp.stack([f.reshape(b, t), c.reshape(b, t)], axis=2)
